# Initial kernel scaffold; baseline (speedup 1.0000x reference)
import jax
import jax.numpy as jnp
from jax.experimental import pallas as pl


def kernel(query_features, history_buffer, W_q, W_k, W_v, W_o, current_step):
    raise NotImplementedError("write your pallas kernel here")


# speedup vs baseline: 2.8114x; 2.8114x over previous
import functools
import math

import jax
import jax.numpy as jnp
from jax.experimental import pallas as pl
from jax.experimental.pallas import tpu as pltpu

_F32 = jnp.float32
_PREC = None

FEATURE_DIM = 256
ATTN_DIM = 128
K_NEIGH = 16
BLK_Q = 128
BLK_KEYS = 2048


def _dot(a, b):
    return jax.lax.dot_general(a, b, (((1,), (0,)), ((), ())),
                               precision=_PREC, preferred_element_type=_F32)


def _dot_t(a, b):
    return jax.lax.dot_general(a, b, (((1,), (1,)), ((), ())),
                               precision=_PREC, preferred_element_type=_F32)


def _keys_kernel(x_ref, wk_ref, wv_ref, kn_ref, k_ref, v_ref):
    x = x_ref[...]
    n = jnp.sqrt(jnp.sum(x * x, axis=1, keepdims=True))
    kn_ref[...] = x / jnp.maximum(n, 1e-12)
    k_ref[...] = _dot(x, wk_ref[...])
    v_ref[...] = _dot(x, wv_ref[...])


def _attn_kernel(scale, thw, hw, cs_ref, q_ref, kn_ref, ka_ref, va_ref,
                 wq_ref, wo_ref, out_ref, cnt_ref):
    q = q_ref[...]
    n = jnp.sqrt(jnp.sum(q * q, axis=1, keepdims=True))
    qn = q / jnp.maximum(n, 1e-12)
    sim = _dot_t(qn, kn_ref[...])

    col = jax.lax.broadcasted_iota(jnp.int32, (BLK_Q, thw), 1)
    big = jnp.int32(thw + 1)

    work = sim
    for _ in range(K_NEIGH):
        m = jnp.max(work, axis=1, keepdims=True)
        idx = jnp.min(jnp.where(work == m, col, big), axis=1, keepdims=True)
        work = jnp.where(col == idx, -3.0, work)
    sel = work < -2.0

    cs = cs_ref[0]
    key_time = col // hw
    keep = sel & (sim > 0.0) & (cs - key_time >= 1) & (key_time < cs)

    qp = _dot(q, wq_ref[...])
    scores = _dot_t(qp, ka_ref[...]) * scale
    scores = jnp.where(keep, scores, -1e9)
    m = jnp.max(scores, axis=1, keepdims=True)
    p = jnp.exp(scores - m)
    attn = p / jnp.sum(p, axis=1, keepdims=True)
    attn = jnp.where(keep, attn, 0.0)

    out = _dot(attn, va_ref[...])
    out_ref[...] = _dot(out, wo_ref[...])
    cnt_ref[...] = jnp.sum(keep.astype(jnp.int32), axis=1, keepdims=True)


def kernel(query_features, history_buffer, W_q, W_k, W_v, W_o, current_step):
    H, W, D = query_features.shape
    T = history_buffer.shape[0]
    HW = H * W
    THW = T * HW
    A = W_q.shape[1]

    keys = history_buffer.reshape(THW, D)
    qf = query_features.reshape(HW, D)

    kn, K_all, V_all = pl.pallas_call(
        _keys_kernel,
        grid=(THW // BLK_KEYS,),
        in_specs=[
            pl.BlockSpec((BLK_KEYS, D), lambda i: (i, 0)),
            pl.BlockSpec((D, A), lambda i: (0, 0)),
            pl.BlockSpec((D, A), lambda i: (0, 0)),
        ],
        out_specs=[
            pl.BlockSpec((BLK_KEYS, D), lambda i: (i, 0)),
            pl.BlockSpec((BLK_KEYS, A), lambda i: (i, 0)),
            pl.BlockSpec((BLK_KEYS, A), lambda i: (i, 0)),
        ],
        out_shape=[
            jax.ShapeDtypeStruct((THW, D), _F32),
            jax.ShapeDtypeStruct((THW, A), _F32),
            jax.ShapeDtypeStruct((THW, A), _F32),
        ],
    )(keys, W_k, W_v)

    scale = 1.0 / (math.sqrt(A) * 1.0)
    cs_arr = jnp.asarray(current_step, jnp.int32).reshape(1)

    out, cnt = pl.pallas_call(
        functools.partial(_attn_kernel, scale, THW, HW),
        grid=(HW // BLK_Q,),
        in_specs=[
            pl.BlockSpec(memory_space=pltpu.SMEM),
            pl.BlockSpec((BLK_Q, D), lambda i: (i, 0)),
            pl.BlockSpec((THW, D), lambda i: (0, 0)),
            pl.BlockSpec((THW, A), lambda i: (0, 0)),
            pl.BlockSpec((THW, A), lambda i: (0, 0)),
            pl.BlockSpec((D, A), lambda i: (0, 0)),
            pl.BlockSpec((A, D), lambda i: (0, 0)),
        ],
        out_specs=[
            pl.BlockSpec((BLK_Q, D), lambda i: (i, 0)),
            pl.BlockSpec((BLK_Q, 1), lambda i: (i, 0)),
        ],
        out_shape=[
            jax.ShapeDtypeStruct((HW, D), _F32),
            jax.ShapeDtypeStruct((HW, 1), jnp.int32),
        ],
    )(cs_arr, qf, kn, K_all, V_all, W_q, W_o)

    output = out.reshape(H, W, D)
    num_connections = jnp.sum(cnt)
    return output, num_connections


# speedup vs baseline: 6.1124x; 2.1741x over previous
import functools
import math

import jax
import jax.numpy as jnp
from jax.experimental import pallas as pl
from jax.experimental.pallas import tpu as pltpu

_F32 = jnp.float32
_PREC = None

K_NEIGH = 16
BLK_Q = 128
BLK_KEYS = 2048


def _dot(a, b):
    return jax.lax.dot_general(a, b, (((1,), (0,)), ((), ())),
                               precision=_PREC, preferred_element_type=_F32)


def _dot_t(a, b):
    return jax.lax.dot_general(a, b, (((1,), (1,)), ((), ())),
                               precision=_PREC, preferred_element_type=_F32)


def _keys_kernel(x_ref, wk_ref, wv_ref, kn_ref, k_ref, v_ref):
    x = x_ref[...]
    n = jnp.sqrt(jnp.sum(x * x, axis=1, keepdims=True))
    kn_ref[...] = x / jnp.maximum(n, 1e-12)
    k_ref[...] = _dot(x, wk_ref[...])
    v_ref[...] = _dot(x, wv_ref[...])


def _attn_tail(scale, hw, thw, cs, sel, sim, q, ka_ref, va_ref, wq_ref,
               wo_ref, out_ref, cnt_ref):
    kt8 = jax.lax.broadcasted_iota(jnp.int32, (8, thw), 1)
    tval = (kt8 < cs * hw)[0:1, :]
    keep = sel & (sim > 0.0) & tval

    qp = _dot(q, wq_ref[...])
    scores = _dot_t(qp, ka_ref[...]) * scale
    scores = jnp.where(keep, scores, -1e9)
    m = jnp.max(scores, axis=1, keepdims=True)
    p = jnp.exp(scores - m)
    attn = p / jnp.sum(p, axis=1, keepdims=True)
    attn = jnp.where(keep, attn, 0.0)

    out = _dot(attn, va_ref[...])
    out_ref[...] = _dot(out, wo_ref[...])
    keepf = jnp.sum(jnp.where(keep, 1.0, 0.0), axis=1, keepdims=True)
    cnt_ref[...] = keepf.astype(jnp.int32)


def _attn_fast_kernel(scale, thw, hw, cs_ref, q_ref, kn_ref, ka_ref, va_ref,
                      wq_ref, wo_ref, out_ref, cnt_ref, bad_ref):
    q = q_ref[...]
    n = jnp.sqrt(jnp.sum(q * q, axis=1, keepdims=True))
    qn = q / jnp.maximum(n, 1e-12)
    sim = _dot_t(qn, kn_ref[...])
    cs = cs_ref[0]

    work = sim
    msum = jnp.zeros((BLK_Q, 1), _F32)
    for _ in range(K_NEIGH):
        m = jnp.max(work, axis=1, keepdims=True)
        msum = msum + m
        work = jnp.where(work == m, -3.0, work)
    sel = work < -2.0
    simsum = jnp.sum(sim, axis=1, keepdims=True)
    worksum = jnp.sum(work, axis=1, keepdims=True)
    expected = simsum - msum - 3.0 * K_NEIGH
    badrow = jnp.abs(worksum - expected) > 1.0
    bad_ref[...] = jnp.where(badrow, 1, 0).astype(jnp.int32)

    _attn_tail(scale, hw, thw, cs, sel, sim, q, ka_ref, va_ref, wq_ref,
               wo_ref, out_ref, cnt_ref)


def _attn_exact_kernel(scale, thw, hw, cs_ref, q_ref, kn_ref, ka_ref, va_ref,
                       wq_ref, wo_ref, out_ref, cnt_ref):
    q = q_ref[...]
    n = jnp.sqrt(jnp.sum(q * q, axis=1, keepdims=True))
    qn = q / jnp.maximum(n, 1e-12)
    sim = _dot_t(qn, kn_ref[...])
    cs = cs_ref[0]

    col = jax.lax.broadcasted_iota(jnp.int32, (BLK_Q, thw), 1)
    big = jnp.int32(thw + 1)
    w = sim
    for _ in range(K_NEIGH):
        m = jnp.max(w, axis=1, keepdims=True)
        idx = jnp.min(jnp.where(w == m, col, big), axis=1, keepdims=True)
        w = jnp.where(col == idx, -3.0, w)
    sel = w < -2.0

    _attn_tail(scale, hw, thw, cs, sel, sim, q, ka_ref, va_ref, wq_ref,
               wo_ref, out_ref, cnt_ref)


def kernel(query_features, history_buffer, W_q, W_k, W_v, W_o, current_step):
    H, W, D = query_features.shape
    T = history_buffer.shape[0]
    HW = H * W
    THW = T * HW
    A = W_q.shape[1]

    keys = history_buffer.reshape(THW, D)
    qf = query_features.reshape(HW, D)

    kn, K_all, V_all = pl.pallas_call(
        _keys_kernel,
        grid=(THW // BLK_KEYS,),
        in_specs=[
            pl.BlockSpec((BLK_KEYS, D), lambda i: (i, 0)),
            pl.BlockSpec((D, A), lambda i: (0, 0)),
            pl.BlockSpec((D, A), lambda i: (0, 0)),
        ],
        out_specs=[
            pl.BlockSpec((BLK_KEYS, D), lambda i: (i, 0)),
            pl.BlockSpec((BLK_KEYS, A), lambda i: (i, 0)),
            pl.BlockSpec((BLK_KEYS, A), lambda i: (i, 0)),
        ],
        out_shape=[
            jax.ShapeDtypeStruct((THW, D), _F32),
            jax.ShapeDtypeStruct((THW, A), _F32),
            jax.ShapeDtypeStruct((THW, A), _F32),
        ],
    )(keys, W_k, W_v)

    scale = 1.0 / (math.sqrt(A) * 1.0)
    cs_arr = jnp.asarray(current_step, jnp.int32).reshape(1)

    grid_q = HW // BLK_Q
    common_in_specs = [
        pl.BlockSpec(memory_space=pltpu.SMEM),
        pl.BlockSpec((BLK_Q, D), lambda i: (i, 0)),
        pl.BlockSpec((THW, D), lambda i: (0, 0)),
        pl.BlockSpec((THW, A), lambda i: (0, 0)),
        pl.BlockSpec((THW, A), lambda i: (0, 0)),
        pl.BlockSpec((D, A), lambda i: (0, 0)),
        pl.BlockSpec((A, D), lambda i: (0, 0)),
    ]
    common_out_specs = [
        pl.BlockSpec((BLK_Q, D), lambda i: (i, 0)),
        pl.BlockSpec((BLK_Q, 1), lambda i: (i, 0)),
    ]
    common_out_shape = [
        jax.ShapeDtypeStruct((HW, D), _F32),
        jax.ShapeDtypeStruct((HW, 1), jnp.int32),
    ]
    args = (cs_arr, qf, kn, K_all, V_all, W_q, W_o)

    out_f, cnt_f, badflags = pl.pallas_call(
        functools.partial(_attn_fast_kernel, scale, THW, HW),
        grid=(grid_q,),
        in_specs=common_in_specs,
        out_specs=common_out_specs + [
            pl.BlockSpec((BLK_Q, 1), lambda i: (i, 0))],
        out_shape=common_out_shape + [
            jax.ShapeDtypeStruct((HW, 1), jnp.int32)],
    )(*args)

    def _exact(_):
        return pl.pallas_call(
            functools.partial(_attn_exact_kernel, scale, THW, HW),
            grid=(grid_q,),
            in_specs=common_in_specs,
            out_specs=common_out_specs,
            out_shape=common_out_shape,
        )(*args)

    out, cnt = jax.lax.cond(jnp.sum(badflags) > 0, _exact,
                            lambda _: (out_f, cnt_f), 0)

    output = out.reshape(H, W, D)
    num_connections = jnp.sum(cnt)
    return output, num_connections


# speedup vs baseline: 6.1374x; 1.0041x over previous
import functools
import math

import jax
import jax.numpy as jnp
from jax.experimental import pallas as pl
from jax.experimental.pallas import tpu as pltpu

_F32 = jnp.float32
_PREC = None

K_NEIGH = 16
BLK_Q = 128
BLK_KEYS = 4096


def _dot(a, b):
    return jax.lax.dot_general(a, b, (((1,), (0,)), ((), ())),
                               precision=_PREC, preferred_element_type=_F32)


def _dot_t(a, b):
    return jax.lax.dot_general(a, b, (((1,), (1,)), ((), ())),
                               precision=_PREC, preferred_element_type=_F32)


def _keys_kernel(x_ref, wk_ref, wv_ref, kn_ref, k_ref, v_ref):
    x = x_ref[...]
    n = jnp.sqrt(jnp.sum(x * x, axis=1, keepdims=True))
    kn_ref[...] = x / jnp.maximum(n, 1e-12)
    k_ref[...] = _dot(x, wk_ref[...])
    v_ref[...] = _dot(x, wv_ref[...])


def _attn_tail(scale, hw, thw, cs, sel, sim, q, ka_ref, va_ref, wq_ref,
               wo_ref, out_ref, cnt_ref):
    kt8 = jax.lax.broadcasted_iota(jnp.int32, (8, thw), 1)
    tval = (kt8 < cs * hw)[0:1, :]
    keep = sel & (sim > 0.0) & tval

    qp = _dot(q, wq_ref[...])
    scores = _dot_t(qp, ka_ref[...]) * scale
    scores = jnp.where(keep, scores, -1e9)
    m = jnp.max(scores, axis=1, keepdims=True)
    p = jnp.exp(scores - m)
    attn = p / jnp.sum(p, axis=1, keepdims=True)
    attn = jnp.where(keep, attn, 0.0)

    out = _dot(attn, va_ref[...])
    out_ref[...] = _dot(out, wo_ref[...])
    keepf = jnp.sum(jnp.where(keep, 1.0, 0.0), axis=1, keepdims=True)
    cnt_ref[...] = keepf.astype(jnp.int32)


def _attn_fast_kernel(scale, thw, hw, cs_ref, q_ref, kn_ref, ka_ref, va_ref,
                      wq_ref, wo_ref, out_ref, cnt_ref, bad_ref):
    q = q_ref[...]
    n = jnp.sqrt(jnp.sum(q * q, axis=1, keepdims=True))
    qn = q / jnp.maximum(n, 1e-12)
    sim = _dot_t(qn, kn_ref[...])
    cs = cs_ref[0]

    work = sim
    msum = jnp.zeros((BLK_Q, 1), _F32)
    for _ in range(K_NEIGH):
        m = jnp.max(work, axis=1, keepdims=True)
        msum = msum + m
        work = jnp.where(work == m, -3.0, work)
    sel = work < -2.0
    simsum = jnp.sum(sim, axis=1, keepdims=True)
    worksum = jnp.sum(work, axis=1, keepdims=True)
    expected = simsum - msum - 3.0 * K_NEIGH
    badrow = jnp.abs(worksum - expected) > 1.0
    bad_ref[...] = jnp.where(badrow, 1, 0).astype(jnp.int32)

    _attn_tail(scale, hw, thw, cs, sel, sim, q, ka_ref, va_ref, wq_ref,
               wo_ref, out_ref, cnt_ref)


def _attn_exact_kernel(scale, thw, hw, cs_ref, q_ref, kn_ref, ka_ref, va_ref,
                       wq_ref, wo_ref, out_ref, cnt_ref):
    q = q_ref[...]
    n = jnp.sqrt(jnp.sum(q * q, axis=1, keepdims=True))
    qn = q / jnp.maximum(n, 1e-12)
    sim = _dot_t(qn, kn_ref[...])
    cs = cs_ref[0]

    col = jax.lax.broadcasted_iota(jnp.int32, (BLK_Q, thw), 1)
    big = jnp.int32(thw + 1)
    w = sim
    for _ in range(K_NEIGH):
        m = jnp.max(w, axis=1, keepdims=True)
        idx = jnp.min(jnp.where(w == m, col, big), axis=1, keepdims=True)
        w = jnp.where(col == idx, -3.0, w)
    sel = w < -2.0

    _attn_tail(scale, hw, thw, cs, sel, sim, q, ka_ref, va_ref, wq_ref,
               wo_ref, out_ref, cnt_ref)


def kernel(query_features, history_buffer, W_q, W_k, W_v, W_o, current_step):
    H, W, D = query_features.shape
    T = history_buffer.shape[0]
    HW = H * W
    THW = T * HW
    A = W_q.shape[1]

    keys = history_buffer.reshape(THW, D)
    qf = query_features.reshape(HW, D)

    kn, K_all, V_all = pl.pallas_call(
        _keys_kernel,
        grid=(THW // BLK_KEYS,),
        in_specs=[
            pl.BlockSpec((BLK_KEYS, D), lambda i: (i, 0)),
            pl.BlockSpec((D, A), lambda i: (0, 0)),
            pl.BlockSpec((D, A), lambda i: (0, 0)),
        ],
        out_specs=[
            pl.BlockSpec((BLK_KEYS, D), lambda i: (i, 0)),
            pl.BlockSpec((BLK_KEYS, A), lambda i: (i, 0)),
            pl.BlockSpec((BLK_KEYS, A), lambda i: (i, 0)),
        ],
        out_shape=[
            jax.ShapeDtypeStruct((THW, D), _F32),
            jax.ShapeDtypeStruct((THW, A), _F32),
            jax.ShapeDtypeStruct((THW, A), _F32),
        ],
    )(keys, W_k, W_v)

    scale = 1.0 / (math.sqrt(A) * 1.0)
    cs_arr = jnp.asarray(current_step, jnp.int32).reshape(1)

    grid_q = HW // BLK_Q
    common_in_specs = [
        pl.BlockSpec(memory_space=pltpu.SMEM),
        pl.BlockSpec((BLK_Q, D), lambda i: (i, 0)),
        pl.BlockSpec((THW, D), lambda i: (0, 0)),
        pl.BlockSpec((THW, A), lambda i: (0, 0)),
        pl.BlockSpec((THW, A), lambda i: (0, 0)),
        pl.BlockSpec((D, A), lambda i: (0, 0)),
        pl.BlockSpec((A, D), lambda i: (0, 0)),
    ]
    common_out_specs = [
        pl.BlockSpec((BLK_Q, D), lambda i: (i, 0)),
        pl.BlockSpec((BLK_Q, 1), lambda i: (i, 0)),
    ]
    common_out_shape = [
        jax.ShapeDtypeStruct((HW, D), _F32),
        jax.ShapeDtypeStruct((HW, 1), jnp.int32),
    ]
    args = (cs_arr, qf, kn, K_all, V_all, W_q, W_o)

    out_f, cnt_f, badflags = pl.pallas_call(
        functools.partial(_attn_fast_kernel, scale, THW, HW),
        grid=(grid_q,),
        in_specs=common_in_specs,
        out_specs=common_out_specs + [
            pl.BlockSpec((BLK_Q, 1), lambda i: (i, 0))],
        out_shape=common_out_shape + [
            jax.ShapeDtypeStruct((HW, 1), jnp.int32)],
    )(*args)

    def _exact(_):
        return pl.pallas_call(
            functools.partial(_attn_exact_kernel, scale, THW, HW),
            grid=(grid_q,),
            in_specs=common_in_specs,
            out_specs=common_out_specs,
            out_shape=common_out_shape,
        )(*args)

    out, cnt = jax.lax.cond(jnp.sum(badflags) > 0, _exact,
                            lambda _: (out_f, cnt_f), 0)

    output = out.reshape(H, W, D)
    num_connections = jnp.sum(cnt)
    return output, num_connections


# speedup vs baseline: 6.2114x; 1.0121x over previous
import functools
import math

import jax
import jax.numpy as jnp
from jax.experimental import pallas as pl
from jax.experimental.pallas import tpu as pltpu

_F32 = jnp.float32
_PREC = None

K_NEIGH = 16
BLK_Q = 128
BLK_KEYS = 4096


def _dot(a, b):
    return jax.lax.dot_general(a, b, (((1,), (0,)), ((), ())),
                               precision=_PREC, preferred_element_type=_F32)


def _dot_t(a, b):
    return jax.lax.dot_general(a, b, (((1,), (1,)), ((), ())),
                               precision=_PREC, preferred_element_type=_F32)


def _keys_kernel(x_ref, wk_ref, wv_ref, kn_ref, k_ref, v_ref):
    x = x_ref[...]
    n = jnp.sqrt(jnp.sum(x * x, axis=1, keepdims=True))
    kn_ref[...] = x / jnp.maximum(n, 1e-12)
    k_ref[...] = _dot(x, wk_ref[...])
    v_ref[...] = _dot(x, wv_ref[...])


def _attn_tail(scale, hw, thw, cs, sel, sim, q, ka_ref, va_ref, wq_ref,
               wo_ref, out_ref, cnt_ref):
    kt8 = jax.lax.broadcasted_iota(jnp.int32, (8, thw), 1)
    tval = (kt8 < cs * hw)[0:1, :]
    keep = sel & (sim > 0.0) & tval

    qp = _dot(q, wq_ref[...]) * scale
    scores = _dot_t(qp, ka_ref[...])
    scores = jnp.where(keep, scores, -1e9)
    m = jnp.max(scores, axis=1, keepdims=True)
    p = jnp.exp(scores - m)
    attn = p / jnp.sum(p, axis=1, keepdims=True)
    attn = jnp.where(keep, attn, 0.0)

    out = _dot(attn, va_ref[...])
    out_ref[...] = _dot(out, wo_ref[...])
    keepf = jnp.sum(jnp.where(keep, 1.0, 0.0), axis=1, keepdims=True)
    cnt_ref[...] = keepf.astype(jnp.int32)


def _attn_fast_kernel(scale, thw, hw, cs_ref, q_ref, kn_ref, ka_ref, va_ref,
                      wq_ref, wo_ref, out_ref, cnt_ref, bad_ref):
    q = q_ref[...]
    n = jnp.sqrt(jnp.sum(q * q, axis=1, keepdims=True))
    qn = q / jnp.maximum(n, 1e-12)
    sim = _dot_t(qn, kn_ref[...])
    cs = cs_ref[0]

    work = sim
    msum = jnp.zeros((BLK_Q, 1), _F32)
    for _ in range(K_NEIGH):
        m = jnp.max(work, axis=1, keepdims=True)
        msum = msum + m
        work = jnp.where(work == m, -3.0, work)
    sel = work < -2.0
    simsum = jnp.sum(sim, axis=1, keepdims=True)
    worksum = jnp.sum(work, axis=1, keepdims=True)
    expected = simsum - msum - 3.0 * K_NEIGH
    badrow = jnp.abs(worksum - expected) > 1.0
    bad_ref[...] = jnp.where(badrow, 1, 0).astype(jnp.int32)

    _attn_tail(scale, hw, thw, cs, sel, sim, q, ka_ref, va_ref, wq_ref,
               wo_ref, out_ref, cnt_ref)


def _attn_exact_kernel(scale, thw, hw, cs_ref, q_ref, kn_ref, ka_ref, va_ref,
                       wq_ref, wo_ref, out_ref, cnt_ref):
    q = q_ref[...]
    n = jnp.sqrt(jnp.sum(q * q, axis=1, keepdims=True))
    qn = q / jnp.maximum(n, 1e-12)
    sim = _dot_t(qn, kn_ref[...])
    cs = cs_ref[0]

    col = jax.lax.broadcasted_iota(jnp.int32, (BLK_Q, thw), 1)
    big = jnp.int32(thw + 1)
    w = sim
    for _ in range(K_NEIGH):
        m = jnp.max(w, axis=1, keepdims=True)
        idx = jnp.min(jnp.where(w == m, col, big), axis=1, keepdims=True)
        w = jnp.where(col == idx, -3.0, w)
    sel = w < -2.0

    _attn_tail(scale, hw, thw, cs, sel, sim, q, ka_ref, va_ref, wq_ref,
               wo_ref, out_ref, cnt_ref)


def kernel(query_features, history_buffer, W_q, W_k, W_v, W_o, current_step):
    H, W, D = query_features.shape
    T = history_buffer.shape[0]
    HW = H * W
    THW = T * HW
    A = W_q.shape[1]

    keys = history_buffer.reshape(THW, D)
    qf = query_features.reshape(HW, D)

    kn, K_all, V_all = pl.pallas_call(
        _keys_kernel,
        grid=(THW // BLK_KEYS,),
        in_specs=[
            pl.BlockSpec((BLK_KEYS, D), lambda i: (i, 0)),
            pl.BlockSpec((D, A), lambda i: (0, 0)),
            pl.BlockSpec((D, A), lambda i: (0, 0)),
        ],
        out_specs=[
            pl.BlockSpec((BLK_KEYS, D), lambda i: (i, 0)),
            pl.BlockSpec((BLK_KEYS, A), lambda i: (i, 0)),
            pl.BlockSpec((BLK_KEYS, A), lambda i: (i, 0)),
        ],
        out_shape=[
            jax.ShapeDtypeStruct((THW, D), _F32),
            jax.ShapeDtypeStruct((THW, A), _F32),
            jax.ShapeDtypeStruct((THW, A), _F32),
        ],
    )(keys, W_k, W_v)

    scale = 1.0 / (math.sqrt(A) * 1.0)
    cs_arr = jnp.asarray(current_step, jnp.int32).reshape(1)

    grid_q = HW // BLK_Q
    common_in_specs = [
        pl.BlockSpec(memory_space=pltpu.SMEM),
        pl.BlockSpec((BLK_Q, D), lambda i: (i, 0)),
        pl.BlockSpec((THW, D), lambda i: (0, 0)),
        pl.BlockSpec((THW, A), lambda i: (0, 0)),
        pl.BlockSpec((THW, A), lambda i: (0, 0)),
        pl.BlockSpec((D, A), lambda i: (0, 0)),
        pl.BlockSpec((A, D), lambda i: (0, 0)),
    ]
    common_out_specs = [
        pl.BlockSpec((BLK_Q, D), lambda i: (i, 0)),
        pl.BlockSpec((BLK_Q, 1), lambda i: (i, 0)),
    ]
    common_out_shape = [
        jax.ShapeDtypeStruct((HW, D), _F32),
        jax.ShapeDtypeStruct((HW, 1), jnp.int32),
    ]
    args = (cs_arr, qf, kn, K_all, V_all, W_q, W_o)

    out_f, cnt_f, badflags = pl.pallas_call(
        functools.partial(_attn_fast_kernel, scale, THW, HW),
        grid=(grid_q,),
        in_specs=common_in_specs,
        out_specs=common_out_specs + [
            pl.BlockSpec((BLK_Q, 1), lambda i: (i, 0))],
        out_shape=common_out_shape + [
            jax.ShapeDtypeStruct((HW, 1), jnp.int32)],
    )(*args)

    def _exact(_):
        return pl.pallas_call(
            functools.partial(_attn_exact_kernel, scale, THW, HW),
            grid=(grid_q,),
            in_specs=common_in_specs,
            out_specs=common_out_specs,
            out_shape=common_out_shape,
        )(*args)

    out, cnt = jax.lax.cond(jnp.sum(badflags) > 0, _exact,
                            lambda _: (out_f, cnt_f), 0)

    output = out.reshape(H, W, D)
    num_connections = jnp.sum(cnt)
    return output, num_connections
